# baseline (speedup 1.0000x reference)
import functools

import jax
import jax.numpy as jnp
from jax import lax
from jax.experimental import pallas as pl
from jax.experimental.pallas import tpu as pltpu
from jax.experimental.pallas import tpu_sc as plsc

B = 16384
D = 768
C = 5
NC = 2
NS = 16
NW = NC * NS
BPW = B // NW
CHUNK = 32
NCHUNK = BPW // CHUNK
RB = 8
NRB = CHUNK // RB
NJ = D // 16


def _shuf(x, idx):
    dn = lax.GatherDimensionNumbers(offset_dims=(), collapsed_slice_dims=(0,),
                                    start_index_map=(0,))
    return lax.gather(x, idx[:, None], dn, (1,),
                      mode=lax.GatherScatterMode.PROMISE_IN_BOUNDS)


def _fused_sc(user_ids, item_ids, user_table, item_table, wt, btile):
    mesh = plsc.VectorSubcoreMesh(core_axis_name="c", subcore_axis_name="s")

    @functools.partial(
        pl.kernel,
        mesh=mesh,
        out_type=jax.ShapeDtypeStruct((B * C,), jnp.float32),
        scratch_types=[
            pltpu.VMEM((BPW,), jnp.int32),
            pltpu.VMEM((BPW,), jnp.int32),
            pltpu.VMEM((2, CHUNK, D), jnp.float32),
            pltpu.VMEM((2, CHUNK, D), jnp.float32),
            pltpu.VMEM((C * 2 * D,), jnp.float32),
            pltpu.VMEM((CHUNK * C,), jnp.float32),
            pltpu.VMEM((2 * CHUNK * C,), jnp.float32),
            pltpu.VMEM((CHUNK * C * 16,), jnp.float32),
            [pltpu.SemaphoreType.DMA] * 2,
            [pltpu.SemaphoreType.DMA] * 2,
            [pltpu.SemaphoreType.DMA] * 2,
        ],
    )
    def k(uid_hbm, iid_hbm, utab_hbm, itab_hbm, wt_hbm, bt_hbm, out_hbm,
          uidx, iidx, ubuf, ibuf, wt_v, bt_v, stage, acc_buf, gsem_u, gsem_i,
          osem):
        wid = lax.axis_index("s") * NC + lax.axis_index("c")
        base = wid * BPW
        pltpu.sync_copy(uid_hbm.at[pl.ds(base, BPW)], uidx)
        pltpu.sync_copy(iid_hbm.at[pl.ds(base, BPW)], iidx)
        pltpu.sync_copy(wt_hbm, wt_v)
        pltpu.sync_copy(bt_hbm, bt_v)

        def start_gather(c, bsl):
            off = pl.multiple_of(c * CHUNK, CHUNK)
            pltpu.async_copy(utab_hbm.at[uidx.at[pl.ds(off, CHUNK)]],
                             ubuf.at[bsl], gsem_u[bsl])
            pltpu.async_copy(itab_hbm.at[iidx.at[pl.ds(off, CHUNK)]],
                             ibuf.at[bsl], gsem_i[bsl])

        def wait_gather(bsl):
            pltpu.make_async_copy(utab_hbm.at[uidx.at[pl.ds(0, CHUNK)]],
                                  ubuf.at[bsl], gsem_u[bsl]).wait()
            pltpu.make_async_copy(itab_hbm.at[iidx.at[pl.ds(0, CHUNK)]],
                                  ibuf.at[bsl], gsem_i[bsl]).wait()

        def start_out(c, bsl):
            off = pl.multiple_of((base + c * CHUNK) * C, CHUNK * C)
            pltpu.async_copy(stage.at[pl.ds(bsl * CHUNK * C, CHUNK * C)],
                             out_hbm.at[pl.ds(off, CHUNK * C)], osem[bsl])

        def wait_out(bsl):
            pltpu.make_async_copy(stage.at[pl.ds(bsl * CHUNK * C, CHUNK * C)],
                                  out_hbm.at[pl.ds(0, CHUNK * C)],
                                  osem[bsl]).wait()

        lane = lax.iota(jnp.int32, 16)
        perms = [jnp.bitwise_xor(lane, d) for d in (1, 2, 4, 8)]

        def compute_chunk(bsl):
            def rb_body(rb, _):
                row0 = rb * RB

                def make_jbody(buf, col0):
                    def jbody(j, accs):
                        j16 = pl.multiple_of(j * 16, 16)
                        ws = [wt_v[pl.ds(cc * 2 * D + col0 + j16, 16)]
                              for cc in range(C)]
                        new = list(accs)
                        for r in range(RB):
                            x = buf[bsl, row0 + r, pl.ds(j16, 16)]
                            for cc in range(C):
                                new[r * C + cc] = new[r * C + cc] + x * ws[cc]
                        return tuple(new)
                    return jbody

                zeros = tuple(jnp.zeros((16,), jnp.float32)
                              for _ in range(RB * C))
                accs = lax.fori_loop(0, NJ, make_jbody(ubuf, 0), zeros)
                accs = lax.fori_loop(0, NJ, make_jbody(ibuf, D), accs)
                for m in range(RB * C):
                    acc_buf[pl.ds((rb * RB * C + m) * 16, 16)] = accs[m]
                return 0

            lax.fori_loop(0, NRB, rb_body, 0)
            for kk in range(CHUNK * C // 16):
                vs = [acc_buf[pl.ds((16 * kk + l) * 16, 16)]
                      for l in range(16)]
                for d_i, d in enumerate((1, 2, 4, 8)):
                    nxt = []
                    for p in range(0, len(vs), 2):
                        fx = vs[p] + _shuf(vs[p], perms[d_i])
                        fy = vs[p + 1] + _shuf(vs[p + 1], perms[d_i])
                        nxt.append(jnp.where((lane & d) == 0, fx, fy))
                    vs = nxt
                v = vs[0] + bt_v[pl.ds(16 * kk, 16)]
                stage[pl.ds(bsl * CHUNK * C + 16 * kk, 16)] = v

        start_gather(0, 0)
        start_gather(1, 1)

        def gbody(g, _):
            for par in range(2):
                c = g * 2 + par
                wait_gather(par)

                @pl.when(c >= 2)
                def _():
                    wait_out(par)

                compute_chunk(par)

                @pl.when(c + 2 < NCHUNK)
                def _():
                    start_gather(c + 2, par)

                start_out(c, par)
            return 0

        lax.fori_loop(0, NCHUNK // 2, gbody, 0)
        wait_out(0)
        wait_out(1)

    return k(user_ids, item_ids, user_table, item_table, wt, btile)


def kernel(user_ids, item_ids, user_table, item_table, W, b):
    uids = user_ids.astype(jnp.int32)
    iids = item_ids.astype(jnp.int32)
    wt = W.T.reshape(-1)
    btile = jnp.tile(b, CHUNK)
    out_flat = _fused_sc(uids, iids, user_table, item_table, wt, btile)
    return out_flat.reshape(B, C)
